# Initial kernel scaffold; baseline (speedup 1.0000x reference)
import jax
import jax.numpy as jnp
from jax.experimental import pallas as pl


def kernel(h_node, pos_node, batch_node, edge_index, batch_edge, t, W_nemb, W_eemb, We1, be1, We2, be2, Wm, bm, Wn1, bn1, Wn2, bn2, Wf1, bf1, Wf2, bf2):
    raise NotImplementedError("write your pallas kernel here")


# speedup vs baseline: 4.4010x; 4.4010x over previous
import functools

import jax
import jax.numpy as jnp
from jax import lax
from jax.experimental import pallas as pl
from jax.experimental.pallas import tpu as pltpu
from jax.experimental.pallas import tpu_sc as plsc

N = 10000; E = 320000; G = 64; T = 16; ND = 128; ED = 128; NB = 3
NP = 10240
NC = 2
NS = 16
NW = NC * NS
CH = 40
TE = 1280
TN = 2000

F32 = jnp.float32


def _mesh():
    return plsc.VectorSubcoreMesh(core_axis_name="c", subcore_axis_name="s",
                                  num_cores=NC, num_subcores=NS)


def _make_sc_gather(M, D, K):
    per_w = M // NW
    nch = per_w // CH
    ngrp = nch // K
    assert ngrp % 2 == 0

    @functools.partial(
        pl.kernel,
        out_type=jax.ShapeDtypeStruct((M, D), F32),
        mesh=_mesh(),
        scratch_types=[pltpu.VMEM((nch, CH), jnp.int32),
                       pltpu.VMEM((2, K * CH, D), F32),
                       pltpu.SemaphoreType.DMA,
                       pltpu.SemaphoreType.DMA],
    )
    def g(tab, idxh, out, idxv, rows, sem0, sem1):
        c = lax.axis_index("c")
        s = lax.axis_index("s")
        w = s * NC + c
        pltpu.sync_copy(idxh.at[w], idxv)
        sems = (sem0, sem1)

        def fire(gi, b):
            for kk in range(K):
                pltpu.async_copy(tab.at[idxv.at[gi * K + kk]],
                                 rows.at[b, pl.ds(kk * CH, CH)], sems[b])

        def drain(gi, b):
            for kk in range(K):
                pltpu.make_async_copy(tab.at[idxv.at[gi * K + kk]],
                                      rows.at[b, pl.ds(kk * CH, CH)],
                                      sems[b]).wait()

        fire(0, 0)

        def outer(go, carry):
            for b in range(2):
                gi = 2 * go + b

                @pl.when(gi + 1 < ngrp)
                def _next():
                    fire(gi + 1, 1 - b)

                drain(gi, b)
                pltpu.sync_copy(
                    rows.at[b],
                    out.at[pl.ds(w * per_w + gi * (K * CH), K * CH)])
            return carry

        lax.fori_loop(0, ngrp // 2, outer, 0)

    return g


def _make_sc_scatter():
    per_w = E // NW
    nch = per_w // CH
    assert nch % 2 == 0
    rpt = NP // NS

    @functools.partial(
        pl.kernel,
        out_type=jax.ShapeDtypeStruct((NC, NP, ND), F32),
        mesh=_mesh(),
        scratch_types=[pltpu.VMEM((nch, CH), jnp.int32),
                       pltpu.VMEM((2, CH, ND), F32),
                       pltpu.VMEM_SHARED((NP, ND), F32),
                       pltpu.SemaphoreType.DMA,
                       pltpu.SemaphoreType.DMA],
    )
    def sc(msg, idxh, zer, out, idxv, slab, table, sem0, sem1):
        c = lax.axis_index("c")
        s = lax.axis_index("s")
        w = s * NC + c
        pltpu.sync_copy(zer.at[pl.ds(s * rpt, rpt)],
                        table.at[pl.ds(s * rpt, rpt)])
        pltpu.sync_copy(idxh.at[w], idxv)
        plsc.subcore_barrier()
        sems = (sem0, sem1)

        def fire(ci, b):
            pltpu.async_copy(msg.at[pl.ds(w * per_w + ci * CH, CH)],
                             slab.at[b], sems[b])

        def drain(ci, b):
            pltpu.make_async_copy(msg.at[pl.ds(w * per_w + ci * CH, CH)],
                                  slab.at[b], sems[b]).wait()

        fire(0, 0)

        def outer(go, carry):
            for b in range(2):
                ci = 2 * go + b

                @pl.when(ci + 1 < nch)
                def _next():
                    fire(ci + 1, 1 - b)

                drain(ci, b)
                pltpu.sync_copy(slab.at[b], table.at[idxv.at[ci]], add=True)
            return carry

        lax.fori_loop(0, nch // 2, outer, 0)
        plsc.subcore_barrier()
        pltpu.sync_copy(table.at[pl.ds(s * rpt, rpt)],
                        out.at[c, pl.ds(s * rpt, rpt)])

    return sc


def _embed_body(hn_ref, w_ref, out_ref):
    out_ref[...] = jnp.dot(hn_ref[...], w_ref[...],
                           preferred_element_type=F32)


def _embed(h_node, W_nemb):
    return pl.pallas_call(
        _embed_body,
        grid=(N // TN,),
        in_specs=[pl.BlockSpec((TN, T), lambda i: (i, 0)),
                  pl.BlockSpec((T, ND), lambda i: (0, 0))],
        out_specs=pl.BlockSpec((TN, ND), lambda i: (i, 0)),
        out_shape=jax.ShapeDtypeStruct((N, ND), F32),
    )(h_node, W_nemb)


def _prologue_body(gs_ref, gd_ref, ws_ref, wd_ref, he_ref, dist_ref):
    gs = gs_ref[...]
    gd = gd_ref[...]
    he = (jnp.dot(gs[:, ND:ND + T], ws_ref[...], preferred_element_type=F32)
          + jnp.dot(gd[:, ND:ND + T], wd_ref[...], preferred_element_type=F32))
    he_ref[...] = he
    d = gs[:, ND + T:ND + T + 3] - gd[:, ND + T:ND + T + 3]
    dist_ref[...] = jnp.sqrt(jnp.sum(d * d, axis=1, keepdims=True) + 1e-8)


def _prologue(g0, Wes, Wed):
    return pl.pallas_call(
        _prologue_body,
        grid=(E // TE,),
        in_specs=[pl.BlockSpec((TE, 256), lambda i: (i, 0)),
                  pl.BlockSpec((TE, 256), lambda i: (E // TE + i, 0)),
                  pl.BlockSpec((T, ED), lambda i: (0, 0)),
                  pl.BlockSpec((T, ED), lambda i: (0, 0))],
        out_specs=[pl.BlockSpec((TE, ED), lambda i: (i, 0)),
                   pl.BlockSpec((TE, 1), lambda i: (i, 0))],
        out_shape=[jax.ShapeDtypeStruct((E, ED), F32),
                   jax.ShapeDtypeStruct((E, 1), F32)],
    )(g0, g0, Wes, Wed)


def _edge_core(he_ref, hs_ref, hd_ref, dist_ref, ae_ref, as_ref, ad_ref,
               adist_ref, be1_ref, w2_ref, be2_ref, wmn_ref, wme_ref, bm_ref):
    he = he_ref[...]
    hs = hs_ref[...]
    hd = hd_ref[...]
    pre = (jnp.dot(he, ae_ref[...], preferred_element_type=F32)
           + jnp.dot(hs, as_ref[...], preferred_element_type=F32)
           + jnp.dot(hd, ad_ref[...], preferred_element_type=F32)
           + dist_ref[...] * adist_ref[...]
           + be1_ref[...])
    h1 = jax.nn.relu(pre)
    heo = he + jnp.dot(h1, w2_ref[...], preferred_element_type=F32) + be2_ref[...]
    msg = jax.nn.relu(jnp.dot(hs, wmn_ref[...], preferred_element_type=F32)
                      + jnp.dot(heo, wme_ref[...], preferred_element_type=F32)
                      + bm_ref[...])
    return heo, msg


def _edge_body(he_ref, hs_ref, hd_ref, dist_ref, ae_ref, as_ref, ad_ref,
               adist_ref, be1_ref, w2_ref, be2_ref, wmn_ref, wme_ref, bm_ref,
               heo_ref, msg_ref):
    heo, msg = _edge_core(he_ref, hs_ref, hd_ref, dist_ref, ae_ref, as_ref,
                          ad_ref, adist_ref, be1_ref, w2_ref, be2_ref,
                          wmn_ref, wme_ref, bm_ref)
    heo_ref[...] = heo
    msg_ref[...] = msg


def _edge_last_body(he_ref, hs_ref, hd_ref, dist_ref, ae_ref, as_ref, ad_ref,
                    adist_ref, be1_ref, w2_ref, be2_ref, wmn_ref, wme_ref,
                    bm_ref, beb_ref, heo_ref, msg_ref, esum_ref, ecnt_ref):
    heo, msg = _edge_core(he_ref, hs_ref, hd_ref, dist_ref, ae_ref, as_ref,
                          ad_ref, adist_ref, be1_ref, w2_ref, be2_ref,
                          wmn_ref, wme_ref, bm_ref)
    heo_ref[...] = heo
    msg_ref[...] = msg
    i = pl.program_id(0)
    be = beb_ref[0, 0, :]
    oh = (lax.broadcasted_iota(jnp.int32, (G, TE), 0)
          == be[None, :]).astype(F32)
    es = jnp.dot(oh, heo, preferred_element_type=F32)
    ec = jnp.sum(oh, axis=1, keepdims=True)

    @pl.when(i == 0)
    def _init():
        esum_ref[...] = es
        ecnt_ref[...] = ec

    @pl.when(i > 0)
    def _acc():
        esum_ref[...] += es
        ecnt_ref[...] += ec


_W_SPEC = [pl.BlockSpec((ND, ND), lambda i: (0, 0)),
           pl.BlockSpec((ND, ND), lambda i: (0, 0)),
           pl.BlockSpec((ND, ND), lambda i: (0, 0)),
           pl.BlockSpec((1, ND), lambda i: (0, 0)),
           pl.BlockSpec((1, ND), lambda i: (0, 0)),
           pl.BlockSpec((ND, ND), lambda i: (0, 0)),
           pl.BlockSpec((1, ND), lambda i: (0, 0)),
           pl.BlockSpec((ND, ND), lambda i: (0, 0)),
           pl.BlockSpec((ND, ND), lambda i: (0, 0)),
           pl.BlockSpec((1, ND), lambda i: (0, 0))]


def _edge_block(h_e, gb, dist, *w):
    return pl.pallas_call(
        _edge_body,
        grid=(E // TE,),
        in_specs=[pl.BlockSpec((TE, ED), lambda i: (i, 0)),
                  pl.BlockSpec((TE, ND), lambda i: (i, 0)),
                  pl.BlockSpec((TE, ND), lambda i: (E // TE + i, 0)),
                  pl.BlockSpec((TE, 1), lambda i: (i, 0))] + _W_SPEC,
        out_specs=[pl.BlockSpec((TE, ED), lambda i: (i, 0)),
                   pl.BlockSpec((TE, ND), lambda i: (i, 0))],
        out_shape=[jax.ShapeDtypeStruct((E, ED), F32),
                   jax.ShapeDtypeStruct((E, ND), F32)],
    )(h_e, gb, gb, dist, *w)


def _edge_block_last(h_e, gb, dist, beb, *w):
    return pl.pallas_call(
        _edge_last_body,
        grid=(E // TE,),
        in_specs=[pl.BlockSpec((TE, ED), lambda i: (i, 0)),
                  pl.BlockSpec((TE, ND), lambda i: (i, 0)),
                  pl.BlockSpec((TE, ND), lambda i: (E // TE + i, 0)),
                  pl.BlockSpec((TE, 1), lambda i: (i, 0))] + _W_SPEC
                 + [pl.BlockSpec((1, 1, TE), lambda i: (i, 0, 0))],
        out_specs=[pl.BlockSpec((TE, ED), lambda i: (i, 0)),
                   pl.BlockSpec((TE, ND), lambda i: (i, 0)),
                   pl.BlockSpec((G, ND), lambda i: (0, 0)),
                   pl.BlockSpec((G, 1), lambda i: (0, 0))],
        out_shape=[jax.ShapeDtypeStruct((E, ED), F32),
                   jax.ShapeDtypeStruct((E, ND), F32),
                   jax.ShapeDtypeStruct((G, ND), F32),
                   jax.ShapeDtypeStruct((G, 1), F32)],
    )(h_e, gb, gb, dist, *w, beb)


def _node_body(hn_ref, a0_ref, a1_ref, w1a_ref, w1b_ref, b1_ref, w2_ref,
               b2_ref, out_ref):
    hn = hn_ref[...]
    a = a0_ref[0] + a1_ref[0]
    pre = (jnp.dot(hn, w1a_ref[...], preferred_element_type=F32)
           + jnp.dot(a, w1b_ref[...], preferred_element_type=F32)
           + b1_ref[...])
    h = jax.nn.relu(pre)
    out_ref[...] = hn + jnp.dot(h, w2_ref[...],
                                preferred_element_type=F32) + b2_ref[...]


def _node_update(h_n, aggp, W1a, W1b, b1, W2, b2):
    return pl.pallas_call(
        _node_body,
        grid=(N // TN,),
        in_specs=[pl.BlockSpec((TN, ND), lambda i: (i, 0)),
                  pl.BlockSpec((1, TN, ND), lambda i: (0, i, 0)),
                  pl.BlockSpec((1, TN, ND), lambda i: (1, i, 0)),
                  pl.BlockSpec((ND, ND), lambda i: (0, 0)),
                  pl.BlockSpec((ND, ND), lambda i: (0, 0)),
                  pl.BlockSpec((1, ND), lambda i: (0, 0)),
                  pl.BlockSpec((ND, ND), lambda i: (0, 0)),
                  pl.BlockSpec((1, ND), lambda i: (0, 0))],
        out_specs=pl.BlockSpec((TN, ND), lambda i: (i, 0)),
        out_shape=jax.ShapeDtypeStruct((N, ND), F32),
    )(h_n, aggp, aggp, W1a, W1b, b1, W2, b2)


def _final_body(hn_ref, bnb_ref, esum_ref, ecnt_ref, wf1_ref, bf1_ref,
                wf2_ref, bf2_ref, pred_ref, nsum_ref, ncnt_ref):
    i = pl.program_id(0)
    bn = bnb_ref[0, 0, :]
    oh = (lax.broadcasted_iota(jnp.int32, (G, TN), 0)
          == bn[None, :]).astype(F32)
    ns = jnp.dot(oh, hn_ref[...], preferred_element_type=F32)
    nc = jnp.sum(oh, axis=1, keepdims=True)

    @pl.when(i == 0)
    def _init():
        nsum_ref[...] = ns
        ncnt_ref[...] = nc

    @pl.when(i > 0)
    def _acc():
        nsum_ref[...] += ns
        ncnt_ref[...] += nc

    @pl.when(i == (N // TN) - 1)
    def _head():
        mean_n = nsum_ref[...] / jnp.maximum(ncnt_ref[...], 1.0)
        mean_e = esum_ref[...] / jnp.maximum(ecnt_ref[...], 1.0)
        h_sub = jnp.concatenate([mean_n, mean_e], axis=1)
        h1 = jax.nn.relu(jnp.dot(h_sub, wf1_ref[...],
                                 preferred_element_type=F32) + bf1_ref[...])
        pred_ref[...] = jnp.dot(h1, wf2_ref[...],
                                preferred_element_type=F32) + bf2_ref[...]


def _final(h_n, bnb, esum, ecnt, Wf1, bf1, Wf2, bf2):
    return pl.pallas_call(
        _final_body,
        grid=(N // TN,),
        in_specs=[pl.BlockSpec((TN, ND), lambda i: (i, 0)),
                  pl.BlockSpec((1, 1, TN), lambda i: (i, 0, 0)),
                  pl.BlockSpec((G, ND), lambda i: (0, 0)),
                  pl.BlockSpec((G, 1), lambda i: (0, 0)),
                  pl.BlockSpec((ND + ED, ND + ED), lambda i: (0, 0)),
                  pl.BlockSpec((1, ND + ED), lambda i: (0, 0)),
                  pl.BlockSpec((ND + ED, 1), lambda i: (0, 0)),
                  pl.BlockSpec((1, 1), lambda i: (0, 0))],
        out_specs=pl.BlockSpec((G, 1), lambda i: (0, 0)),
        out_shape=jax.ShapeDtypeStruct((G, 1), F32),
        scratch_shapes=[pltpu.VMEM((G, ND), F32), pltpu.VMEM((G, 1), F32)],
    )(h_n, bnb, esum, ecnt, Wf1, bf1, Wf2, bf2)


_sc_gather256 = None
_sc_gather128 = None
_sc_scatter = None


def _get_sc():
    global _sc_gather256, _sc_gather128, _sc_scatter
    if _sc_gather256 is None:
        _sc_gather256 = _make_sc_gather(2 * E, 256, 2)
        _sc_gather128 = _make_sc_gather(2 * E, ND, 2)
        _sc_scatter = _make_sc_scatter()
    return _sc_gather256, _sc_gather128, _sc_scatter


def kernel(h_node, pos_node, batch_node, edge_index, batch_edge, t,
           W_nemb, W_eemb, We1, be1, We2, be2, Wm, bm, Wn1, bn1, Wn2, bn2,
           Wf1, bf1, Wf2, bf2):
    sc_g256, sc_g128, sc_scat = _get_sc()

    idx2 = edge_index.reshape(NW, 2 * E // (NW * CH), CH)
    dstr = edge_index[1].reshape(NW, E // (NW * CH), CH)
    zer = jnp.zeros((NP, ND), F32)
    beb = batch_edge.reshape(E // TE, 1, TE)
    bnb = batch_node.reshape(N // TN, 1, TN)

    h_n = _embed(h_node, W_nemb)
    packed = jnp.concatenate(
        [h_n, h_node, pos_node, jnp.zeros((N, 256 - ND - T - 3), F32)],
        axis=1)
    g0 = sc_g256(packed, idx2)
    h_e, dist = _prologue(g0, W_eemb[:T], W_eemb[T:])

    esum = ecnt = None
    for b in range(NB):
        gb = g0 if b == 0 else sc_g128(h_n, idx2)
        w = (We1[b, :ED], We1[b, ED:ED + ND], We1[b, ED + ND:ED + 2 * ND],
             We1[b, ED + 2 * ND:ED + 2 * ND + 1], be1[b].reshape(1, ND),
             We2[b], be2[b].reshape(1, ND),
             Wm[b, :ND], Wm[b, ND:], bm[b].reshape(1, ND))
        if b < NB - 1:
            h_e, msg = _edge_block(h_e, gb, dist, *w)
        else:
            h_e, msg, esum, ecnt = _edge_block_last(h_e, gb, dist, beb, *w)
        aggp = sc_scat(msg, dstr, zer)
        h_n = _node_update(h_n, aggp, Wn1[b, :ND], Wn1[b, ND:2 * ND],
                           bn1[b].reshape(1, ND), Wn2[b],
                           bn2[b].reshape(1, ND))

    return _final(h_n, bnb, esum, ecnt, Wf1, bf1.reshape(1, ND + ED),
                  Wf2, bf2.reshape(1, 1))


# speedup vs baseline: 5.2315x; 1.1887x over previous
import functools

import jax
import jax.numpy as jnp
from jax import lax
from jax.experimental import pallas as pl
from jax.experimental.pallas import tpu as pltpu
from jax.experimental.pallas import tpu_sc as plsc

N = 10000; E = 320000; G = 64; T = 16; ND = 128; ED = 128; NB = 3
NP = 10240
EC = E // 2
NC = 2
NS = 16
NW = NC * NS
CH = 40
CHG = 80
TE = 1280
TN = 2000

F32 = jnp.float32
BF16 = jnp.bfloat16


def _mesh():
    return plsc.VectorSubcoreMesh(core_axis_name="c", subcore_axis_name="s",
                                  num_cores=NC, num_subcores=NS)


def _make_sc_gather(M, D, K, dtype=F32, ch=CHG):
    per_w = M // NW
    nch = per_w // ch
    ngrp = nch // K
    assert ngrp % 2 == 0

    @functools.partial(
        pl.kernel,
        out_type=jax.ShapeDtypeStruct((M, D), dtype),
        mesh=_mesh(),
        scratch_types=[pltpu.VMEM((nch, ch), jnp.int32),
                       pltpu.VMEM((2, K * ch, D), dtype),
                       pltpu.SemaphoreType.DMA,
                       pltpu.SemaphoreType.DMA],
    )
    def g(tab, idxh, out, idxv, rows, sem0, sem1):
        c = lax.axis_index("c")
        s = lax.axis_index("s")
        w = s * NC + c
        pltpu.sync_copy(idxh.at[w], idxv)
        sems = (sem0, sem1)

        def fire(gi, b):
            for kk in range(K):
                pltpu.async_copy(tab.at[idxv.at[gi * K + kk]],
                                 rows.at[b, pl.ds(kk * ch, ch)], sems[b])

        def drain(gi, b):
            for kk in range(K):
                pltpu.make_async_copy(tab.at[idxv.at[gi * K + kk]],
                                      rows.at[b, pl.ds(kk * ch, ch)],
                                      sems[b]).wait()

        fire(0, 0)

        def outer(go, carry):
            for b in range(2):
                gi = 2 * go + b

                @pl.when(gi + 1 < ngrp)
                def _next():
                    fire(gi + 1, 1 - b)

                drain(gi, b)
                pltpu.sync_copy(
                    rows.at[b],
                    out.at[pl.ds(w * per_w + gi * (K * ch), K * ch)])
            return carry

        lax.fori_loop(0, ngrp // 2, outer, 0)

    return g


def _make_sc_scatter(M=E):
    per_w = M // NW
    nch = per_w // CH
    rpt = NP // NS

    @functools.partial(
        pl.kernel,
        out_type=jax.ShapeDtypeStruct((NC, NP, ND), F32),
        mesh=_mesh(),
        scratch_types=[pltpu.VMEM((nch, CH), jnp.int32),
                       pltpu.VMEM((2, CH, ND), F32),
                       pltpu.VMEM_SHARED((NP, ND), F32),
                       pltpu.SemaphoreType.DMA,
                       pltpu.SemaphoreType.DMA],
    )
    def sc(msg, idxh, zer, out, idxv, slab, table, sem0, sem1):
        c = lax.axis_index("c")
        s = lax.axis_index("s")
        w = s * NC + c
        pltpu.sync_copy(zer.at[pl.ds(s * rpt, rpt)],
                        table.at[pl.ds(s * rpt, rpt)])
        pltpu.sync_copy(idxh.at[w], idxv)
        plsc.subcore_barrier()
        sems = (sem0, sem1)

        def fire(ci, b):
            pltpu.async_copy(msg.at[pl.ds(w * per_w + ci * CH, CH)],
                             slab.at[b], sems[b])

        def drain(ci, b):
            pltpu.make_async_copy(msg.at[pl.ds(w * per_w + ci * CH, CH)],
                                  slab.at[b], sems[b]).wait()

        fire(0, 0)

        def outer(go, carry):
            for b in range(2):
                ci = 2 * go + b

                @pl.when(ci + 1 < nch)
                def _next():
                    fire(ci + 1, 1 - b)

                drain(ci, b)
                pltpu.sync_copy(slab.at[b], table.at[idxv.at[ci]], add=True)
            return carry

        lax.fori_loop(0, nch // 2, outer, 0)
        if nch % 2 == 1:
            drain(nch - 1, 0)
            pltpu.sync_copy(slab.at[0], table.at[idxv.at[nch - 1]], add=True)
        plsc.subcore_barrier()
        pltpu.sync_copy(table.at[pl.ds(s * rpt, rpt)],
                        out.at[c, pl.ds(s * rpt, rpt)])

    return sc


def _embed_body(hn_ref, w_ref, out_ref):
    out_ref[...] = jnp.dot(hn_ref[...], w_ref[...],
                           preferred_element_type=F32)


def _embed(h_node, W_nemb):
    return pl.pallas_call(
        _embed_body,
        grid=(N // TN,),
        in_specs=[pl.BlockSpec((TN, T), lambda i: (i, 0)),
                  pl.BlockSpec((T, ND), lambda i: (0, 0))],
        out_specs=pl.BlockSpec((TN, ND), lambda i: (i, 0)),
        out_shape=jax.ShapeDtypeStruct((N, ND), F32),
    )(h_node, W_nemb)


def _edge0_body(gs_ref, gd_ref, ws_ref, wd_ref, ae_ref, as_ref, ad_ref,
                adist_ref, be1_ref, w2_ref, be2_ref, wmn_ref, wme_ref,
                bm_ref, heo_ref, msg_ref, dist_ref):
    gs = gs_ref[...]
    gd = gd_ref[...]
    he = (jnp.dot(gs[:, ND:ND + T], ws_ref[...], preferred_element_type=F32)
          + jnp.dot(gd[:, ND:ND + T], wd_ref[...], preferred_element_type=F32))
    d = gs[:, ND + T:ND + T + 3] - gd[:, ND + T:ND + T + 3]
    dist = jnp.sqrt(jnp.sum(d * d, axis=1, keepdims=True) + 1e-8)
    dist_ref[...] = dist
    hs = gs[:, :ND]
    hd = gd[:, :ND]
    pre = (jnp.dot(he, ae_ref[...], preferred_element_type=F32)
           + jnp.dot(hs, as_ref[...], preferred_element_type=F32)
           + jnp.dot(hd, ad_ref[...], preferred_element_type=F32)
           + dist * adist_ref[...]
           + be1_ref[...])
    h1 = jax.nn.relu(pre)
    heo = he + jnp.dot(h1, w2_ref[...], preferred_element_type=F32) + be2_ref[...]
    heo_ref[...] = heo
    msg_ref[...] = jax.nn.relu(
        jnp.dot(hs, wmn_ref[...], preferred_element_type=F32)
        + jnp.dot(heo, wme_ref[...], preferred_element_type=F32)
        + bm_ref[...])


def _edge0_block(g0c, Wes, Wed, *w):
    return pl.pallas_call(
        _edge0_body,
        grid=(EC // TE,),
        in_specs=[pl.BlockSpec((TE, 256), lambda i: (i, 0)),
                  pl.BlockSpec((TE, 256), lambda i: (EC // TE + i, 0)),
                  pl.BlockSpec((T, ED), lambda i: (0, 0)),
                  pl.BlockSpec((T, ED), lambda i: (0, 0))] + _W_SPEC,
        out_specs=[pl.BlockSpec((TE, ED), lambda i: (i, 0)),
                   pl.BlockSpec((TE, ND), lambda i: (i, 0)),
                   pl.BlockSpec((TE, 1), lambda i: (i, 0))],
        out_shape=[jax.ShapeDtypeStruct((EC, ED), F32),
                   jax.ShapeDtypeStruct((EC, ND), F32),
                   jax.ShapeDtypeStruct((EC, 1), F32)],
    )(g0c, g0c, Wes, Wed, *w)


def _edge_core(he_ref, hs_ref, hd_ref, dist_ref, ae_ref, as_ref, ad_ref,
               adist_ref, be1_ref, w2_ref, be2_ref, wmn_ref, wme_ref, bm_ref):
    he = he_ref[...]
    hs = hs_ref[...]
    hd = hd_ref[...]
    pre = (jnp.dot(he, ae_ref[...], preferred_element_type=F32)
           + jnp.dot(hs, as_ref[...], preferred_element_type=F32)
           + jnp.dot(hd, ad_ref[...], preferred_element_type=F32)
           + dist_ref[...] * adist_ref[...]
           + be1_ref[...])
    h1 = jax.nn.relu(pre)
    heo = he + jnp.dot(h1, w2_ref[...], preferred_element_type=F32) + be2_ref[...]
    msg = jax.nn.relu(jnp.dot(hs, wmn_ref[...], preferred_element_type=F32)
                      + jnp.dot(heo, wme_ref[...], preferred_element_type=F32)
                      + bm_ref[...])
    return heo, msg


def _edge_body(he_ref, hs_ref, hd_ref, dist_ref, ae_ref, as_ref, ad_ref,
               adist_ref, be1_ref, w2_ref, be2_ref, wmn_ref, wme_ref, bm_ref,
               heo_ref, msg_ref):
    heo, msg = _edge_core(he_ref, hs_ref, hd_ref, dist_ref, ae_ref, as_ref,
                          ad_ref, adist_ref, be1_ref, w2_ref, be2_ref,
                          wmn_ref, wme_ref, bm_ref)
    heo_ref[...] = heo
    msg_ref[...] = msg


def _edge_last_body(he_ref, hs_ref, hd_ref, dist_ref, ae_ref, as_ref, ad_ref,
                    adist_ref, be1_ref, w2_ref, be2_ref, wmn_ref, wme_ref,
                    bm_ref, beb_ref, heo_ref, msg_ref, esum_ref, ecnt_ref):
    heo, msg = _edge_core(he_ref, hs_ref, hd_ref, dist_ref, ae_ref, as_ref,
                          ad_ref, adist_ref, be1_ref, w2_ref, be2_ref,
                          wmn_ref, wme_ref, bm_ref)
    heo_ref[...] = heo
    msg_ref[...] = msg
    i = pl.program_id(0)
    be = beb_ref[0, 0, :]
    oh = (lax.broadcasted_iota(jnp.int32, (G, TE), 0)
          == be[None, :]).astype(F32)
    es = jnp.dot(oh, heo, preferred_element_type=F32)
    ec = jnp.sum(oh, axis=1, keepdims=True)

    @pl.when(i == 0)
    def _init():
        esum_ref[...] = es
        ecnt_ref[...] = ec

    @pl.when(i > 0)
    def _acc():
        esum_ref[...] += es
        ecnt_ref[...] += ec


_W_SPEC = [pl.BlockSpec((ND, ND), lambda i: (0, 0)),
           pl.BlockSpec((ND, ND), lambda i: (0, 0)),
           pl.BlockSpec((ND, ND), lambda i: (0, 0)),
           pl.BlockSpec((1, ND), lambda i: (0, 0)),
           pl.BlockSpec((1, ND), lambda i: (0, 0)),
           pl.BlockSpec((ND, ND), lambda i: (0, 0)),
           pl.BlockSpec((1, ND), lambda i: (0, 0)),
           pl.BlockSpec((ND, ND), lambda i: (0, 0)),
           pl.BlockSpec((ND, ND), lambda i: (0, 0)),
           pl.BlockSpec((1, ND), lambda i: (0, 0))]


def _edge_block(h_e, gb, dist, *w):
    return pl.pallas_call(
        _edge_body,
        grid=(EC // TE,),
        in_specs=[pl.BlockSpec((TE, ED), lambda i: (i, 0)),
                  pl.BlockSpec((TE, ND), lambda i: (i, 0)),
                  pl.BlockSpec((TE, ND), lambda i: (EC // TE + i, 0)),
                  pl.BlockSpec((TE, 1), lambda i: (i, 0))] + _W_SPEC,
        out_specs=[pl.BlockSpec((TE, ED), lambda i: (i, 0)),
                   pl.BlockSpec((TE, ND), lambda i: (i, 0))],
        out_shape=[jax.ShapeDtypeStruct((EC, ED), F32),
                   jax.ShapeDtypeStruct((EC, ND), F32)],
    )(h_e, gb, gb, dist, *w)


def _edge_block_last(h_e, gb, dist, beb, *w):
    return pl.pallas_call(
        _edge_last_body,
        grid=(EC // TE,),
        in_specs=[pl.BlockSpec((TE, ED), lambda i: (i, 0)),
                  pl.BlockSpec((TE, ND), lambda i: (i, 0)),
                  pl.BlockSpec((TE, ND), lambda i: (EC // TE + i, 0)),
                  pl.BlockSpec((TE, 1), lambda i: (i, 0))] + _W_SPEC
                 + [pl.BlockSpec((1, 1, TE), lambda i: (i, 0, 0))],
        out_specs=[pl.BlockSpec((TE, ED), lambda i: (i, 0)),
                   pl.BlockSpec((TE, ND), lambda i: (i, 0)),
                   pl.BlockSpec((G, ND), lambda i: (0, 0)),
                   pl.BlockSpec((G, 1), lambda i: (0, 0))],
        out_shape=[jax.ShapeDtypeStruct((EC, ED), F32),
                   jax.ShapeDtypeStruct((EC, ND), F32),
                   jax.ShapeDtypeStruct((G, ND), F32),
                   jax.ShapeDtypeStruct((G, 1), F32)],
    )(h_e, gb, gb, dist, *w, beb)


def _node_body(hn_ref, a00_ref, a01_ref, a10_ref, a11_ref, w1a_ref, w1b_ref,
               b1_ref, w2_ref, b2_ref, out_ref):
    hn = hn_ref[...]
    a = (a00_ref[0] + a01_ref[0]) + (a10_ref[0] + a11_ref[0])
    pre = (jnp.dot(hn, w1a_ref[...], preferred_element_type=F32)
           + jnp.dot(a, w1b_ref[...], preferred_element_type=F32)
           + b1_ref[...])
    h = jax.nn.relu(pre)
    out_ref[...] = hn + jnp.dot(h, w2_ref[...],
                                preferred_element_type=F32) + b2_ref[...]


def _node_update(h_n, agg0, agg1, W1a, W1b, b1, W2, b2):
    aspec = [pl.BlockSpec((1, TN, ND), lambda i: (0, i, 0)),
             pl.BlockSpec((1, TN, ND), lambda i: (1, i, 0))]
    return pl.pallas_call(
        _node_body,
        grid=(N // TN,),
        in_specs=[pl.BlockSpec((TN, ND), lambda i: (i, 0))]
                 + aspec + aspec
                 + [pl.BlockSpec((ND, ND), lambda i: (0, 0)),
                    pl.BlockSpec((ND, ND), lambda i: (0, 0)),
                    pl.BlockSpec((1, ND), lambda i: (0, 0)),
                    pl.BlockSpec((ND, ND), lambda i: (0, 0)),
                    pl.BlockSpec((1, ND), lambda i: (0, 0))],
        out_specs=pl.BlockSpec((TN, ND), lambda i: (i, 0)),
        out_shape=jax.ShapeDtypeStruct((N, ND), F32),
    )(h_n, agg0, agg0, agg1, agg1, W1a, W1b, b1, W2, b2)


def _final_body(hn_ref, bnb_ref, es0_ref, ec0_ref, es1_ref, ec1_ref,
                wf1_ref, bf1_ref, wf2_ref, bf2_ref, pred_ref,
                nsum_ref, ncnt_ref):
    i = pl.program_id(0)
    bn = bnb_ref[0, 0, :]
    oh = (lax.broadcasted_iota(jnp.int32, (G, TN), 0)
          == bn[None, :]).astype(F32)
    ns = jnp.dot(oh, hn_ref[...], preferred_element_type=F32)
    nc = jnp.sum(oh, axis=1, keepdims=True)

    @pl.when(i == 0)
    def _init():
        nsum_ref[...] = ns
        ncnt_ref[...] = nc

    @pl.when(i > 0)
    def _acc():
        nsum_ref[...] += ns
        ncnt_ref[...] += nc

    @pl.when(i == (N // TN) - 1)
    def _head():
        mean_n = nsum_ref[...] / jnp.maximum(ncnt_ref[...], 1.0)
        esum = es0_ref[...] + es1_ref[...]
        ecnt = ec0_ref[...] + ec1_ref[...]
        mean_e = esum / jnp.maximum(ecnt, 1.0)
        h_sub = jnp.concatenate([mean_n, mean_e], axis=1)
        h1 = jax.nn.relu(jnp.dot(h_sub, wf1_ref[...],
                                 preferred_element_type=F32) + bf1_ref[...])
        pred_ref[...] = jnp.dot(h1, wf2_ref[...],
                                preferred_element_type=F32) + bf2_ref[...]


def _final(h_n, bnb, es0, ec0, es1, ec1, Wf1, bf1, Wf2, bf2):
    return pl.pallas_call(
        _final_body,
        grid=(N // TN,),
        in_specs=[pl.BlockSpec((TN, ND), lambda i: (i, 0)),
                  pl.BlockSpec((1, 1, TN), lambda i: (i, 0, 0)),
                  pl.BlockSpec((G, ND), lambda i: (0, 0)),
                  pl.BlockSpec((G, 1), lambda i: (0, 0)),
                  pl.BlockSpec((G, ND), lambda i: (0, 0)),
                  pl.BlockSpec((G, 1), lambda i: (0, 0)),
                  pl.BlockSpec((ND + ED, ND + ED), lambda i: (0, 0)),
                  pl.BlockSpec((1, ND + ED), lambda i: (0, 0)),
                  pl.BlockSpec((ND + ED, 1), lambda i: (0, 0)),
                  pl.BlockSpec((1, 1), lambda i: (0, 0))],
        out_specs=pl.BlockSpec((G, 1), lambda i: (0, 0)),
        out_shape=jax.ShapeDtypeStruct((G, 1), F32),
        scratch_shapes=[pltpu.VMEM((G, ND), F32), pltpu.VMEM((G, 1), F32)],
    )(h_n, bnb, es0, ec0, es1, ec1, Wf1, bf1, Wf2, bf2)


_sc_gather256 = None
_sc_gather128 = None
_sc_scatter = None


def _get_sc():
    global _sc_gather256, _sc_gather128, _sc_scatter
    if _sc_gather256 is None:
        _sc_gather256 = _make_sc_gather(2 * EC, 256, 1, F32, CH)
        _sc_gather128 = _make_sc_gather(2 * EC, ND, 5, F32, CH)
        _sc_scatter = _make_sc_scatter(EC)
    return _sc_gather256, _sc_gather128, _sc_scatter


def kernel(h_node, pos_node, batch_node, edge_index, batch_edge, t,
           W_nemb, W_eemb, We1, be1, We2, be2, Wm, bm, Wn1, bn1, Wn2, bn2,
           Wf1, bf1, Wf2, bf2):
    sc_g256, sc_g128, sc_scat = _get_sc()

    src = edge_index[0]
    dst = edge_index[1]
    idx2c = [jnp.concatenate([src[c * EC:(c + 1) * EC],
                              dst[c * EC:(c + 1) * EC]])
             .reshape(NW, 2 * EC // (NW * CH), CH) for c in range(2)]
    dstrc = [dst[c * EC:(c + 1) * EC].reshape(NW, EC // (NW * CH), CH)
             for c in range(2)]
    zer = jnp.zeros((NP, ND), F32)
    bebc = [batch_edge[c * EC:(c + 1) * EC].reshape(EC // TE, 1, TE)
            for c in range(2)]
    bnb = batch_node.reshape(N // TN, 1, TN)

    h_n = _embed(h_node, W_nemb)
    packed = jnp.concatenate(
        [h_n, h_node, pos_node, jnp.zeros((N, 256 - ND - T - 3), F32)],
        axis=1)

    h_e = [None, None]
    dist = [None, None]
    msg = [None, None]
    esum = [None, None]
    ecnt = [None, None]
    for b in range(NB):
        w = (We1[b, :ED], We1[b, ED:ED + ND], We1[b, ED + ND:ED + 2 * ND],
             We1[b, ED + 2 * ND:ED + 2 * ND + 1], be1[b].reshape(1, ND),
             We2[b], be2[b].reshape(1, ND),
             Wm[b, :ND], Wm[b, ND:], bm[b].reshape(1, ND))
        for c in range(2):
            if b == 0:
                g = sc_g256(packed, idx2c[c])
                h_e[c], msg[c], dist[c] = _edge0_block(
                    g, W_eemb[:T], W_eemb[T:], *w)
            elif b < NB - 1:
                g = sc_g128(h_n, idx2c[c])
                h_e[c], msg[c] = _edge_block(h_e[c], g, dist[c], *w)
            else:
                g = sc_g128(h_n, idx2c[c])
                h_e[c], msg[c], esum[c], ecnt[c] = _edge_block_last(
                    h_e[c], g, dist[c], bebc[c], *w)
        agg = [sc_scat(msg[c], dstrc[c], zer) for c in range(2)]
        h_n = _node_update(h_n, agg[0], agg[1], Wn1[b, :ND],
                           Wn1[b, ND:2 * ND], bn1[b].reshape(1, ND),
                           Wn2[b], bn2[b].reshape(1, ND))

    return _final(h_n, bnb, esum[0], ecnt[0], esum[1], ecnt[1],
                  Wf1, bf1.reshape(1, ND + ED), Wf2, bf2.reshape(1, 1))


# speedup vs baseline: 5.3427x; 1.0212x over previous
import functools

import jax
import jax.numpy as jnp
from jax import lax
from jax.experimental import pallas as pl
from jax.experimental.pallas import tpu as pltpu
from jax.experimental.pallas import tpu_sc as plsc

N = 10000; E = 320000; G = 64; T = 16; ND = 128; ED = 128; NB = 3
NP = 10240
EC = E // 2
NC = 2
NS = 16
NW = NC * NS
CH = 40
CHG = 80
TE = 1280
TN = 2000

F32 = jnp.float32
PREC = jax.lax.Precision.HIGHEST
BF16 = jnp.bfloat16


def _mesh():
    return plsc.VectorSubcoreMesh(core_axis_name="c", subcore_axis_name="s",
                                  num_cores=NC, num_subcores=NS)


def _make_sc_gather(M, D, K, dtype=F32, ch=CHG):
    per_w = M // NW
    nch = per_w // ch
    ngrp = nch // K
    assert ngrp % 2 == 0

    @functools.partial(
        pl.kernel,
        out_type=jax.ShapeDtypeStruct((M, D), dtype),
        mesh=_mesh(),
        scratch_types=[pltpu.VMEM((nch, ch), jnp.int32),
                       pltpu.VMEM((2, K * ch, D), dtype),
                       pltpu.SemaphoreType.DMA,
                       pltpu.SemaphoreType.DMA],
    )
    def g(tab, idxh, out, idxv, rows, sem0, sem1):
        c = lax.axis_index("c")
        s = lax.axis_index("s")
        w = s * NC + c
        pltpu.sync_copy(idxh.at[w], idxv)
        sems = (sem0, sem1)

        def fire(gi, b):
            for kk in range(K):
                pltpu.async_copy(tab.at[idxv.at[gi * K + kk]],
                                 rows.at[b, pl.ds(kk * ch, ch)], sems[b])

        def drain(gi, b):
            for kk in range(K):
                pltpu.make_async_copy(tab.at[idxv.at[gi * K + kk]],
                                      rows.at[b, pl.ds(kk * ch, ch)],
                                      sems[b]).wait()

        fire(0, 0)

        def outer(go, carry):
            for b in range(2):
                gi = 2 * go + b

                @pl.when(gi + 1 < ngrp)
                def _next():
                    fire(gi + 1, 1 - b)

                drain(gi, b)
                pltpu.sync_copy(
                    rows.at[b],
                    out.at[pl.ds(w * per_w + gi * (K * ch), K * ch)])
            return carry

        lax.fori_loop(0, ngrp // 2, outer, 0)

    return g


def _make_sc_gather_staged(M, ch=80):
    per_w = M // NW
    nch = per_w // ch
    rpt = NP // NS

    @functools.partial(
        pl.kernel,
        out_type=jax.ShapeDtypeStruct((M, ND), F32),
        mesh=_mesh(),
        scratch_types=[pltpu.VMEM((nch, ch), jnp.int32),
                       pltpu.VMEM((2, ch, ND), F32),
                       pltpu.VMEM_SHARED((NP, ND), F32),
                       pltpu.SemaphoreType.DMA,
                       pltpu.SemaphoreType.DMA],
    )
    def g(tab, idxh, out, idxv, rows, stab, sem0, sem1):
        c = lax.axis_index("c")
        s = lax.axis_index("s")
        w = s * NC + c
        pltpu.sync_copy(tab.at[pl.ds(s * rpt, rpt)],
                        stab.at[pl.ds(s * rpt, rpt)])
        pltpu.sync_copy(idxh.at[w], idxv)
        plsc.subcore_barrier()
        sems = (sem0, sem1)

        def fire(gi, b):
            pltpu.async_copy(stab.at[idxv.at[gi]], rows.at[b], sems[b])

        def drain(gi, b):
            pltpu.make_async_copy(stab.at[idxv.at[gi]], rows.at[b],
                                  sems[b]).wait()

        fire(0, 0)

        def outer(go, carry):
            for b in range(2):
                gi = 2 * go + b

                @pl.when(gi + 1 < nch)
                def _next():
                    fire(gi + 1, 1 - b)

                drain(gi, b)
                pltpu.sync_copy(rows.at[b],
                                out.at[pl.ds(w * per_w + gi * ch, ch)])
            return carry

        lax.fori_loop(0, nch // 2, outer, 0)
        if nch % 2 == 1:
            drain(nch - 1, 0)
            pltpu.sync_copy(rows.at[0],
                            out.at[pl.ds(w * per_w + (nch - 1) * ch, ch)])

    return g


def _make_sc_scatter(M=E):
    per_w = M // NW
    nch = per_w // CH
    rpt = NP // NS

    @functools.partial(
        pl.kernel,
        out_type=jax.ShapeDtypeStruct((NC, NP, ND), F32),
        mesh=_mesh(),
        scratch_types=[pltpu.VMEM((nch, CH), jnp.int32),
                       pltpu.VMEM((2, CH, ND), F32),
                       pltpu.VMEM_SHARED((NP, ND), F32),
                       pltpu.SemaphoreType.DMA,
                       pltpu.SemaphoreType.DMA],
    )
    def sc(msg, idxh, zer, out, idxv, slab, table, sem0, sem1):
        c = lax.axis_index("c")
        s = lax.axis_index("s")
        w = s * NC + c
        pltpu.sync_copy(zer.at[pl.ds(s * rpt, rpt)],
                        table.at[pl.ds(s * rpt, rpt)])
        pltpu.sync_copy(idxh.at[w], idxv)
        plsc.subcore_barrier()
        sems = (sem0, sem1)

        def fire(ci, b):
            pltpu.async_copy(msg.at[pl.ds(w * per_w + ci * CH, CH)],
                             slab.at[b], sems[b])

        def drain(ci, b):
            pltpu.make_async_copy(msg.at[pl.ds(w * per_w + ci * CH, CH)],
                                  slab.at[b], sems[b]).wait()

        fire(0, 0)

        def outer(go, carry):
            for b in range(2):
                ci = 2 * go + b

                @pl.when(ci + 1 < nch)
                def _next():
                    fire(ci + 1, 1 - b)

                drain(ci, b)
                pltpu.sync_copy(slab.at[b], table.at[idxv.at[ci]], add=True)
            return carry

        lax.fori_loop(0, nch // 2, outer, 0)
        if nch % 2 == 1:
            drain(nch - 1, 0)
            pltpu.sync_copy(slab.at[0], table.at[idxv.at[nch - 1]], add=True)
        plsc.subcore_barrier()
        pltpu.sync_copy(table.at[pl.ds(s * rpt, rpt)],
                        out.at[c, pl.ds(s * rpt, rpt)])

    return sc


def _embed_body(hn_ref, w_ref, out_ref):
    out_ref[...] = jnp.dot(hn_ref[...], w_ref[...],
                           preferred_element_type=F32)


def _embed(h_node, W_nemb):
    return pl.pallas_call(
        _embed_body,
        grid=(N // TN,),
        in_specs=[pl.BlockSpec((TN, T), lambda i: (i, 0)),
                  pl.BlockSpec((T, ND), lambda i: (0, 0))],
        out_specs=pl.BlockSpec((TN, ND), lambda i: (i, 0)),
        out_shape=jax.ShapeDtypeStruct((N, ND), F32),
    )(h_node, W_nemb)


def _core(he, hs, hd, dist, we1_ref, be1_ref, w2_ref, be2_ref, wm_ref,
          bm_ref):
    e_in = jnp.concatenate(
        [he, hs, hd, dist, jnp.zeros((he.shape[0], 1), F32)], axis=1)
    pre = jnp.dot(e_in, we1_ref[...], preferred_element_type=F32) + be1_ref[...]
    h1 = jax.nn.relu(pre)
    heo = he + jnp.dot(h1, w2_ref[...], preferred_element_type=F32) + be2_ref[...]
    m_in = jnp.concatenate([hs, heo], axis=1)
    msg = jax.nn.relu(jnp.dot(m_in, wm_ref[...],
                              preferred_element_type=F32) + bm_ref[...])
    return heo, msg


def _edge0_body(gs_ref, gd_ref, wee_ref, we1_ref, be1_ref, w2_ref, be2_ref,
                wm_ref, bm_ref, heo_ref, msg_ref, dist_ref):
    gs = gs_ref[...]
    gd = gd_ref[...]
    raw = jnp.concatenate([gs[:, ND:ND + T], gd[:, ND:ND + T]], axis=1)
    he = jnp.dot(raw, wee_ref[...], preferred_element_type=F32)
    d = gs[:, ND + T:ND + T + 3] - gd[:, ND + T:ND + T + 3]
    dist = jnp.sqrt(jnp.sum(d * d, axis=1, keepdims=True) + 1e-8)
    dist_ref[...] = dist
    heo, msg = _core(he, gs[:, :ND], gd[:, :ND], dist, we1_ref, be1_ref,
                     w2_ref, be2_ref, wm_ref, bm_ref)
    heo_ref[...] = heo
    msg_ref[...] = msg


def _edge0_block(g0c, Wee, *w):
    return pl.pallas_call(
        _edge0_body,
        grid=(EC // TE,),
        in_specs=[pl.BlockSpec((TE, 256), lambda i: (i, 0)),
                  pl.BlockSpec((TE, 256), lambda i: (EC // TE + i, 0)),
                  pl.BlockSpec((2 * T, ED), lambda i: (0, 0))] + _W_SPEC,
        out_specs=[pl.BlockSpec((TE, ED), lambda i: (i, 0)),
                   pl.BlockSpec((TE, ND), lambda i: (i, 0)),
                   pl.BlockSpec((TE, 1), lambda i: (i, 0))],
        out_shape=[jax.ShapeDtypeStruct((EC, ED), F32),
                   jax.ShapeDtypeStruct((EC, ND), F32),
                   jax.ShapeDtypeStruct((EC, 1), F32)],
    )(g0c, g0c, Wee, *w)


def _edge_body(he_ref, hs_ref, hd_ref, dist_ref, we1_ref, be1_ref, w2_ref,
               be2_ref, wm_ref, bm_ref, heo_ref, msg_ref):
    heo, msg = _core(he_ref[...], hs_ref[...], hd_ref[...], dist_ref[...],
                     we1_ref, be1_ref, w2_ref, be2_ref, wm_ref, bm_ref)
    heo_ref[...] = heo
    msg_ref[...] = msg


def _edge_last_body(he_ref, hs_ref, hd_ref, dist_ref, we1_ref, be1_ref,
                    w2_ref, be2_ref, wm_ref, bm_ref, beb_ref, heo_ref,
                    msg_ref, esum_ref, ecnt_ref):
    heo, msg = _core(he_ref[...], hs_ref[...], hd_ref[...], dist_ref[...],
                     we1_ref, be1_ref, w2_ref, be2_ref, wm_ref, bm_ref)
    heo_ref[...] = heo
    msg_ref[...] = msg
    i = pl.program_id(0)
    be = beb_ref[0, 0, :]
    oh = (lax.broadcasted_iota(jnp.int32, (G, TE), 0)
          == be[None, :]).astype(F32)
    es = jnp.dot(oh, heo, preferred_element_type=F32, precision=PREC)
    ec = jnp.sum(oh, axis=1, keepdims=True)

    @pl.when(i == 0)
    def _init():
        esum_ref[...] = es
        ecnt_ref[...] = ec

    @pl.when(i > 0)
    def _acc():
        esum_ref[...] += es
        ecnt_ref[...] += ec


_W_SPEC = [pl.BlockSpec((ED + 2 * ND + 2, ED), lambda i: (0, 0)),
           pl.BlockSpec((1, ND), lambda i: (0, 0)),
           pl.BlockSpec((ND, ND), lambda i: (0, 0)),
           pl.BlockSpec((1, ND), lambda i: (0, 0)),
           pl.BlockSpec((2 * ND, ND), lambda i: (0, 0)),
           pl.BlockSpec((1, ND), lambda i: (0, 0))]


def _edge_block(h_e, gb, dist, *w):
    return pl.pallas_call(
        _edge_body,
        grid=(EC // TE,),
        in_specs=[pl.BlockSpec((TE, ED), lambda i: (i, 0)),
                  pl.BlockSpec((TE, ND), lambda i: (i, 0)),
                  pl.BlockSpec((TE, ND), lambda i: (EC // TE + i, 0)),
                  pl.BlockSpec((TE, 1), lambda i: (i, 0))] + _W_SPEC,
        out_specs=[pl.BlockSpec((TE, ED), lambda i: (i, 0)),
                   pl.BlockSpec((TE, ND), lambda i: (i, 0))],
        out_shape=[jax.ShapeDtypeStruct((EC, ED), F32),
                   jax.ShapeDtypeStruct((EC, ND), F32)],
    )(h_e, gb, gb, dist, *w)


def _edge_block_last(h_e, gb, dist, beb, *w):
    return pl.pallas_call(
        _edge_last_body,
        grid=(EC // TE,),
        in_specs=[pl.BlockSpec((TE, ED), lambda i: (i, 0)),
                  pl.BlockSpec((TE, ND), lambda i: (i, 0)),
                  pl.BlockSpec((TE, ND), lambda i: (EC // TE + i, 0)),
                  pl.BlockSpec((TE, 1), lambda i: (i, 0))] + _W_SPEC
                 + [pl.BlockSpec((1, 1, TE), lambda i: (i, 0, 0))],
        out_specs=[pl.BlockSpec((TE, ED), lambda i: (i, 0)),
                   pl.BlockSpec((TE, ND), lambda i: (i, 0)),
                   pl.BlockSpec((G, ND), lambda i: (0, 0)),
                   pl.BlockSpec((G, 1), lambda i: (0, 0))],
        out_shape=[jax.ShapeDtypeStruct((EC, ED), F32),
                   jax.ShapeDtypeStruct((EC, ND), F32),
                   jax.ShapeDtypeStruct((G, ND), F32),
                   jax.ShapeDtypeStruct((G, 1), F32)],
    )(h_e, gb, gb, dist, *w, beb)


def _node_body(hn_ref, a00_ref, a01_ref, a10_ref, a11_ref, w1_ref,
               b1_ref, w2_ref, b2_ref, out_ref):
    hn = hn_ref[...]
    a = (a00_ref[0] + a01_ref[0]) + (a10_ref[0] + a11_ref[0])
    n_in = jnp.concatenate([hn, a, jnp.zeros((TN, 1), F32)], axis=1)
    pre = jnp.dot(n_in, w1_ref[...], preferred_element_type=F32) + b1_ref[...]
    h = jax.nn.relu(pre)
    out_ref[...] = hn + jnp.dot(h, w2_ref[...],
                                preferred_element_type=F32) + b2_ref[...]


def _node_update(h_n, agg0, agg1, W1, b1, W2, b2):
    aspec = [pl.BlockSpec((1, TN, ND), lambda i: (0, i, 0)),
             pl.BlockSpec((1, TN, ND), lambda i: (1, i, 0))]
    return pl.pallas_call(
        _node_body,
        grid=(N // TN,),
        in_specs=[pl.BlockSpec((TN, ND), lambda i: (i, 0))]
                 + aspec + aspec
                 + [pl.BlockSpec((2 * ND + 1, ND), lambda i: (0, 0)),
                    pl.BlockSpec((1, ND), lambda i: (0, 0)),
                    pl.BlockSpec((ND, ND), lambda i: (0, 0)),
                    pl.BlockSpec((1, ND), lambda i: (0, 0))],
        out_specs=pl.BlockSpec((TN, ND), lambda i: (i, 0)),
        out_shape=jax.ShapeDtypeStruct((N, ND), F32),
    )(h_n, agg0, agg0, agg1, agg1, W1, b1, W2, b2)


def _final_body(hn_ref, bnb_ref, es0_ref, ec0_ref, es1_ref, ec1_ref,
                wf1_ref, bf1_ref, wf2_ref, bf2_ref, pred_ref,
                nsum_ref, ncnt_ref):
    i = pl.program_id(0)
    bn = bnb_ref[0, 0, :]
    oh = (lax.broadcasted_iota(jnp.int32, (G, TN), 0)
          == bn[None, :]).astype(F32)
    ns = jnp.dot(oh, hn_ref[...], preferred_element_type=F32,
                 precision=PREC)
    nc = jnp.sum(oh, axis=1, keepdims=True)

    @pl.when(i == 0)
    def _init():
        nsum_ref[...] = ns
        ncnt_ref[...] = nc

    @pl.when(i > 0)
    def _acc():
        nsum_ref[...] += ns
        ncnt_ref[...] += nc

    @pl.when(i == (N // TN) - 1)
    def _head():
        mean_n = nsum_ref[...] / jnp.maximum(ncnt_ref[...], 1.0)
        esum = es0_ref[...] + es1_ref[...]
        ecnt = ec0_ref[...] + ec1_ref[...]
        mean_e = esum / jnp.maximum(ecnt, 1.0)
        h_sub = jnp.concatenate([mean_n, mean_e], axis=1)
        h1 = jax.nn.relu(jnp.dot(h_sub, wf1_ref[...],
                                 preferred_element_type=F32) + bf1_ref[...])
        pred_ref[...] = jnp.dot(h1, wf2_ref[...],
                                preferred_element_type=F32) + bf2_ref[...]


def _final(h_n, bnb, es0, ec0, es1, ec1, Wf1, bf1, Wf2, bf2):
    return pl.pallas_call(
        _final_body,
        grid=(N // TN,),
        in_specs=[pl.BlockSpec((TN, ND), lambda i: (i, 0)),
                  pl.BlockSpec((1, 1, TN), lambda i: (i, 0, 0)),
                  pl.BlockSpec((G, ND), lambda i: (0, 0)),
                  pl.BlockSpec((G, 1), lambda i: (0, 0)),
                  pl.BlockSpec((G, ND), lambda i: (0, 0)),
                  pl.BlockSpec((G, 1), lambda i: (0, 0)),
                  pl.BlockSpec((ND + ED, ND + ED), lambda i: (0, 0)),
                  pl.BlockSpec((1, ND + ED), lambda i: (0, 0)),
                  pl.BlockSpec((ND + ED, 1), lambda i: (0, 0)),
                  pl.BlockSpec((1, 1), lambda i: (0, 0))],
        out_specs=pl.BlockSpec((G, 1), lambda i: (0, 0)),
        out_shape=jax.ShapeDtypeStruct((G, 1), F32),
        scratch_shapes=[pltpu.VMEM((G, ND), F32), pltpu.VMEM((G, 1), F32)],
    )(h_n, bnb, es0, ec0, es1, ec1, Wf1, bf1, Wf2, bf2)


_sc_gather256 = None
_sc_gather128 = None
_sc_scatter = None


def _get_sc():
    global _sc_gather256, _sc_gather128, _sc_scatter
    if _sc_gather256 is None:
        _sc_gather256 = _make_sc_gather(2 * EC, 256, 1, F32, CH)
        _sc_gather128 = _make_sc_gather_staged(2 * EC, CHG)
        _sc_scatter = _make_sc_scatter(EC)
    return _sc_gather256, _sc_gather128, _sc_scatter


def kernel(h_node, pos_node, batch_node, edge_index, batch_edge, t,
           W_nemb, W_eemb, We1, be1, We2, be2, Wm, bm, Wn1, bn1, Wn2, bn2,
           Wf1, bf1, Wf2, bf2):
    sc_g256, sc_g128, sc_scat = _get_sc()

    src = edge_index[0]
    dst = edge_index[1]
    idx2c = [jnp.concatenate([src[c * EC:(c + 1) * EC],
                              dst[c * EC:(c + 1) * EC]])
             .reshape(NW, 2 * EC // (NW * CH), CH) for c in range(2)]
    idx2cg = [jnp.concatenate([src[c * EC:(c + 1) * EC],
                               dst[c * EC:(c + 1) * EC]])
              .reshape(NW, 2 * EC // (NW * CHG), CHG) for c in range(2)]
    dstrc = [dst[c * EC:(c + 1) * EC].reshape(NW, EC // (NW * CH), CH)
             for c in range(2)]
    zer = jnp.zeros((NP, ND), F32)
    bebc = [batch_edge[c * EC:(c + 1) * EC].reshape(EC // TE, 1, TE)
            for c in range(2)]
    bnb = batch_node.reshape(N // TN, 1, TN)

    h_n = _embed(h_node, W_nemb)
    packed = jnp.concatenate(
        [h_n, h_node, pos_node, jnp.zeros((N, 256 - ND - T - 3), F32)],
        axis=1)

    h_e = [None, None]
    dist = [None, None]
    msg = [None, None]
    esum = [None, None]
    ecnt = [None, None]
    for b in range(NB):
        h_np = (None if b == 0 else
                jnp.concatenate([h_n, jnp.zeros((NP - N, ND), F32)], axis=0))
        w = (We1[b], be1[b].reshape(1, ND), We2[b], be2[b].reshape(1, ND),
             Wm[b], bm[b].reshape(1, ND))
        for c in range(2):
            if b == 0:
                g = sc_g256(packed, idx2c[c])
                h_e[c], msg[c], dist[c] = _edge0_block(g, W_eemb, *w)
            elif b < NB - 1:
                g = sc_g128(h_np, idx2cg[c])
                h_e[c], msg[c] = _edge_block(h_e[c], g, dist[c], *w)
            else:
                g = sc_g128(h_np, idx2cg[c])
                h_e[c], msg[c], esum[c], ecnt[c] = _edge_block_last(
                    h_e[c], g, dist[c], bebc[c], *w)
        agg = [sc_scat(msg[c], dstrc[c], zer) for c in range(2)]
        h_n = _node_update(h_n, agg[0], agg[1], Wn1[b],
                           bn1[b].reshape(1, ND), Wn2[b],
                           bn2[b].reshape(1, ND))

    return _final(h_n, bnb, esum[0], ecnt[0], esum[1], ecnt[1],
                  Wf1, bf1.reshape(1, ND + ED), Wf2, bf2.reshape(1, 1))
